# Initial kernel scaffold; baseline (speedup 1.0000x reference)
import jax
import jax.numpy as jnp
from jax.experimental import pallas as pl


def kernel(T_n_1, T_n_2, edge_index, edge_vals, theta):
    raise NotImplementedError("write your pallas kernel here")


# speedup vs baseline: 6.6623x; 6.6623x over previous
import functools

import jax
import jax.numpy as jnp
from jax import lax
from jax.experimental import pallas as pl
from jax.experimental.pallas import tpu as pltpu
from jax.experimental.pallas import tpu_sc as plsc

_N = 10000
_E = 320000
_D = 128
_NC = 2
_NS = 16
_NW = _NC * _NS
_L = 16
_EPT = _E // _NW
_CH = 80
_SCCH = 25
_ESC = _SCCH * _CH
_NSC = _EPT // _ESC
_NP = 10240
_RPT = _NP // _NS
_ZCH = 64
_NZCH = _RPT // _ZCH
_RB = 1000
_NRB = _N // _RB

_mesh = plsc.VectorSubcoreMesh(core_axis_name="c", subcore_axis_name="s")


@functools.partial(
    pl.kernel,
    mesh=_mesh,
    out_type=jax.ShapeDtypeStruct((_NC, _NP, _D), jnp.float32),
    scratch_types=[
        pltpu.VMEM((_ESC,), jnp.int32),
        pltpu.VMEM((_SCCH, _CH), jnp.int32),
        pltpu.VMEM((_ESC,), jnp.float32),
        pltpu.VMEM((_CH, _D), jnp.float32),
        pltpu.VMEM((_ZCH, _D), jnp.float32),
        pltpu.VMEM_SHARED((_NP, _D), jnp.float32),
        pltpu.SemaphoreType.DMA,
    ],
)
def _spmm(t1, cols, row3, vals, part,
          colv, rowv, valv, gbuf, zbuf, acc, sem):
    c = lax.axis_index("c")
    s = lax.axis_index("s")
    w = c * _NS + s

    def z_body(i, _):
        for k in range(_D // _L):
            zbuf[i, pl.ds(k * _L, _L)] = jnp.zeros((_L,), jnp.float32)
        return 0

    lax.fori_loop(0, _ZCH, z_body, 0)
    for rc in range(_NZCH):
        pltpu.sync_copy(zbuf, acc.at[pl.ds(s * _RPT + rc * _ZCH, _ZCH)])
    plsc.subcore_barrier()

    def sc_body(t, _0):
        e0 = w * _EPT + t * _ESC
        pltpu.sync_copy(cols.at[pl.ds(e0, _ESC)], colv)
        pltpu.sync_copy(row3.at[w * _NSC + t], rowv)
        pltpu.sync_copy(vals.at[pl.ds(e0, _ESC)], valv)

        def chunk_body(j, _1):
            pltpu.async_copy(
                t1.at[colv.at[pl.ds(j * _CH, _CH)]], gbuf, sem).wait()

            def g_body(g, _2):
                vals16 = valv[pl.ds(j * _CH + g * _L, _L)]
                for jj in range(_L):
                    bc = lax.gather(
                        vals16,
                        jnp.full((_L, 1), jj, jnp.int32),
                        dimension_numbers=lax.GatherDimensionNumbers(
                            offset_dims=(), collapsed_slice_dims=(0,),
                            start_index_map=(0,)),
                        slice_sizes=(1,),
                        mode=lax.GatherScatterMode.PROMISE_IN_BOUNDS)
                    r = g * _L + jj
                    for k in range(_D // _L):
                        sl = pl.ds(k * _L, _L)
                        gbuf[r, sl] = gbuf[r, sl] * bc
                return 0

            lax.fori_loop(0, _CH // _L, g_body, 0)
            pltpu.sync_copy(gbuf, acc.at[rowv.at[j]], add=True)
            return 0

        lax.fori_loop(0, _SCCH, chunk_body, 0)
        return 0

    lax.fori_loop(0, _NSC, sc_body, 0)
    plsc.subcore_barrier()

    pltpu.sync_copy(acc.at[pl.ds(s * _RPT, _RPT)],
                    part.at[c, pl.ds(s * _RPT, _RPT)])


def _combine_body(p_ref, t2_ref, th_ref, h_ref, h2_ref):
    h = 2.0 * (p_ref[0] + p_ref[1]) - t2_ref[...]
    h_ref[...] = h
    h2_ref[...] = th_ref[0, 0] * h


_combine = pl.pallas_call(
    _combine_body,
    grid=(_NRB,),
    in_specs=[
        pl.BlockSpec((_NC, _RB, _D), lambda i: (0, i, 0)),
        pl.BlockSpec((_RB, _D), lambda i: (i, 0)),
        pl.BlockSpec((8, _D), lambda i: (0, 0)),
    ],
    out_specs=[
        pl.BlockSpec((_RB, _D), lambda i: (i, 0)),
        pl.BlockSpec((_RB, _D), lambda i: (i, 0)),
    ],
    out_shape=(
        jax.ShapeDtypeStruct((_N, _D), jnp.float32),
        jax.ShapeDtypeStruct((_N, _D), jnp.float32),
    ),
)


def kernel(T_n_1, T_n_2, edge_index, edge_vals, theta):
    col = edge_index[1]
    row3 = jnp.reshape(edge_index[0], (_NW * _NSC, _SCCH, _CH))
    thb = jnp.broadcast_to(theta.astype(jnp.float32).reshape(1, 1), (8, _D))

    part = _spmm(T_n_1, col, row3, edge_vals)
    H_l, out2 = _combine(part, T_n_2, thb)
    return (H_l, out2)


# speedup vs baseline: 9.6405x; 1.4470x over previous
import functools

import jax
import jax.numpy as jnp
from jax import lax
from jax.experimental import pallas as pl
from jax.experimental.pallas import tpu as pltpu
from jax.experimental.pallas import tpu_sc as plsc

_N = 10000
_E = 320000
_D = 128
_NC = 2
_NS = 16
_NW = _NC * _NS
_L = 16
_EPT = _E // _NW
_CH = 80
_SCCH = 25
_ESC = _SCCH * _CH
_NSC = _EPT // _ESC
_NP = 10240
_RPT = _NP // _NS
_ZCH = 64
_NZCH = _RPT // _ZCH
_RB = 1000
_NRB = _N // _RB

_mesh = plsc.VectorSubcoreMesh(core_axis_name="c", subcore_axis_name="s")


@functools.partial(
    pl.kernel,
    mesh=_mesh,
    out_type=jax.ShapeDtypeStruct((_NC, _NP, _D), jnp.float32),
    scratch_types=[
        pltpu.VMEM((_ESC,), jnp.int32),
        pltpu.VMEM((_SCCH, _CH), jnp.int32),
        pltpu.VMEM((_ESC,), jnp.float32),
        pltpu.VMEM((_CH, _D), jnp.float32),
        pltpu.VMEM((_CH, _D), jnp.float32),
        pltpu.VMEM_SHARED((_NP, _D), jnp.float32),
        pltpu.SemaphoreType.DMA,
        pltpu.SemaphoreType.DMA,
    ],
)
def _spmm(t1, cols, row3, vals, part,
          colv, rowv, valv, g0, g1, acc, sem0, sem1):
    c = lax.axis_index("c")
    s = lax.axis_index("s")
    w = c * _NS + s

    def z_body(i, _):
        for k in range(_D // _L):
            g0[i, pl.ds(k * _L, _L)] = jnp.zeros((_L,), jnp.float32)
        return 0

    lax.fori_loop(0, _CH, z_body, 0)
    for rc in range(_RPT // _CH):
        pltpu.sync_copy(g0, acc.at[pl.ds(s * _RPT + rc * _CH, _CH)])
    plsc.subcore_barrier()

    def start_gather(j, gb, sem):
        pltpu.async_copy(t1.at[colv.at[pl.ds(j * _CH, _CH)]], gb, sem)

    def wait_gather(gb, sem):
        pltpu.make_async_copy(t1.at[pl.ds(0, _CH)], gb, sem).wait()

    def scale(gb, j):
        def g_body(g, _2):
            vals16 = valv[pl.ds(j * _CH + g * _L, _L)]
            for jj in range(_L):
                bc = lax.gather(
                    vals16,
                    jnp.full((_L, 1), jj, jnp.int32),
                    dimension_numbers=lax.GatherDimensionNumbers(
                        offset_dims=(), collapsed_slice_dims=(0,),
                        start_index_map=(0,)),
                    slice_sizes=(1,),
                    mode=lax.GatherScatterMode.PROMISE_IN_BOUNDS)
                r = g * _L + jj
                for k in range(_D // _L):
                    sl = pl.ds(k * _L, _L)
                    gb[r, sl] = gb[r, sl] * bc
            return 0

        lax.fori_loop(0, _CH // _L, g_body, 0)

    def scatter(gb, j):
        pltpu.sync_copy(gb, acc.at[rowv.at[j]], add=True)

    def sc_body(t, _0):
        e0 = w * _EPT + t * _ESC
        pltpu.sync_copy(cols.at[pl.ds(e0, _ESC)], colv)
        pltpu.sync_copy(row3.at[w * _NSC + t], rowv)
        pltpu.sync_copy(vals.at[pl.ds(e0, _ESC)], valv)

        start_gather(0, g0, sem0)

        def pair_body(i, _1):
            j = 2 * i
            wait_gather(g0, sem0)
            start_gather(j + 1, g1, sem1)
            scale(g0, j)
            scatter(g0, j)
            wait_gather(g1, sem1)
            start_gather(j + 2, g0, sem0)
            scale(g1, j + 1)
            scatter(g1, j + 1)
            return 0

        lax.fori_loop(0, (_SCCH - 1) // 2, pair_body, 0)
        wait_gather(g0, sem0)
        scale(g0, _SCCH - 1)
        scatter(g0, _SCCH - 1)
        return 0

    lax.fori_loop(0, _NSC, sc_body, 0)
    plsc.subcore_barrier()

    pltpu.sync_copy(acc.at[pl.ds(s * _RPT, _RPT)],
                    part.at[c, pl.ds(s * _RPT, _RPT)])


def _combine_body(p_ref, t2_ref, th_ref, h_ref, h2_ref):
    h = 2.0 * (p_ref[0] + p_ref[1]) - t2_ref[...]
    h_ref[...] = h
    h2_ref[...] = th_ref[0, 0] * h


_combine = pl.pallas_call(
    _combine_body,
    grid=(_NRB,),
    in_specs=[
        pl.BlockSpec((_NC, _RB, _D), lambda i: (0, i, 0)),
        pl.BlockSpec((_RB, _D), lambda i: (i, 0)),
        pl.BlockSpec((8, _D), lambda i: (0, 0)),
    ],
    out_specs=[
        pl.BlockSpec((_RB, _D), lambda i: (i, 0)),
        pl.BlockSpec((_RB, _D), lambda i: (i, 0)),
    ],
    out_shape=(
        jax.ShapeDtypeStruct((_N, _D), jnp.float32),
        jax.ShapeDtypeStruct((_N, _D), jnp.float32),
    ),
)


def kernel(T_n_1, T_n_2, edge_index, edge_vals, theta):
    col = edge_index[1]
    row3 = jnp.reshape(edge_index[0], (_NW * _NSC, _SCCH, _CH))
    thb = jnp.broadcast_to(theta.astype(jnp.float32).reshape(1, 1), (8, _D))

    part = _spmm(T_n_1, col, row3, edge_vals)
    H_l, out2 = _combine(part, T_n_2, thb)
    return (H_l, out2)


# speedup vs baseline: 11.1014x; 1.1515x over previous
import functools

import jax
import jax.numpy as jnp
from jax import lax
from jax.experimental import pallas as pl
from jax.experimental.pallas import tpu as pltpu
from jax.experimental.pallas import tpu_sc as plsc

_N = 10000
_E = 320000
_D = 128
_NC = 2
_NS = 16
_NW = _NC * _NS
_L = 16
_EPT = _E // _NW
_CH = 80
_SCCH = 25
_ESC = _SCCH * _CH
_NSC = _EPT // _ESC
_NP = 10240
_RPT = _NP // _NS
_ZCH = 64
_NZCH = _RPT // _ZCH
_RB = 1000
_NRB = _N // _RB

_mesh = plsc.VectorSubcoreMesh(core_axis_name="c", subcore_axis_name="s")


@functools.partial(
    pl.kernel,
    mesh=_mesh,
    out_type=jax.ShapeDtypeStruct((_NC, _NP, _D), jnp.float32),
    scratch_types=[
        pltpu.VMEM((_ESC,), jnp.int32),
        pltpu.VMEM((_SCCH, _CH), jnp.int32),
        pltpu.VMEM((_ESC,), jnp.float32),
        pltpu.VMEM((_CH, _D), jnp.float32),
        pltpu.VMEM((_CH, _D), jnp.float32),
        pltpu.VMEM((_CH, _D), jnp.float32),
        pltpu.VMEM_SHARED((_NP, _D), jnp.float32),
        pltpu.SemaphoreType.DMA,
        pltpu.SemaphoreType.DMA,
        pltpu.SemaphoreType.DMA,
        pltpu.SemaphoreType.DMA,
        pltpu.SemaphoreType.DMA,
        pltpu.SemaphoreType.DMA,
    ],
)
def _spmm(t1, cols, row3, vals, part,
          colv, rowv, valv, g0, g1, g2, acc,
          gs0, gs1, gs2, ss0, ss1, ss2):
    c = lax.axis_index("c")
    s = lax.axis_index("s")
    w = c * _NS + s

    def z_body(i, _):
        for k in range(_D // _L):
            g0[i, pl.ds(k * _L, _L)] = jnp.zeros((_L,), jnp.float32)
        return 0

    lax.fori_loop(0, _CH, z_body, 0)
    for rc in range(_RPT // _CH):
        pltpu.sync_copy(g0, acc.at[pl.ds(s * _RPT + rc * _CH, _CH)])
    plsc.subcore_barrier()

    bufs = ((g0, gs0, ss0), (g1, gs1, ss1), (g2, gs2, ss2))

    def start_gather(j, gb, sem):
        pltpu.async_copy(t1.at[colv.at[pl.ds(j * _CH, _CH)]], gb, sem)

    def wait_gather(gb, sem):
        pltpu.make_async_copy(t1.at[pl.ds(0, _CH)], gb, sem).wait()

    def start_scatter(gb, j, sem):
        pltpu.async_copy(gb, acc.at[rowv.at[j]], sem, add=True)

    def wait_scatter(gb, sem):
        pltpu.make_async_copy(gb, acc.at[pl.ds(0, _CH)], sem).wait()

    def scale(gb, j):
        def q_body(q, _2):
            g = q // 4
            vals16 = valv[pl.ds(j * _CH + g * _L, _L)]
            for d in range(4):
                lane = (q % 4) * 4 + d
                bc = lax.gather(
                    vals16,
                    jnp.full((_L, 1), lane, jnp.int32),
                    dimension_numbers=lax.GatherDimensionNumbers(
                        offset_dims=(), collapsed_slice_dims=(0,),
                        start_index_map=(0,)),
                    slice_sizes=(1,),
                    mode=lax.GatherScatterMode.PROMISE_IN_BOUNDS)
                r = q * 4 + d
                for k in range(_D // _L):
                    sl = pl.ds(k * _L, _L)
                    gb[r, sl] = gb[r, sl] * bc
            return 0

        lax.fori_loop(0, _CH // 4, q_body, 0)

    def slot(j, b, first, prefetch):
        buf, gs, ss = bufs[b]
        pbuf, pgs, pss = bufs[(b + 2) % 3]
        wait_gather(buf, gs)
        scale(buf, j)
        start_scatter(buf, j, ss)
        if prefetch:
            if first:
                start_gather(j + 2, pbuf, pgs)
            else:
                @pl.when(j + 2 < _SCCH)
                def _():
                    wait_scatter(pbuf, pss)
                    start_gather(j + 2, pbuf, pgs)

    def sc_body(t, _0):
        e0 = w * _EPT + t * _ESC
        pltpu.sync_copy(cols.at[pl.ds(e0, _ESC)], colv)
        pltpu.sync_copy(row3.at[w * _NSC + t], rowv)
        pltpu.sync_copy(vals.at[pl.ds(e0, _ESC)], valv)

        start_gather(0, g0, gs0)
        start_gather(1, g1, gs1)
        slot(0, 0, True, True)
        slot(1, 1, False, True)
        slot(2, 2, False, True)

        def tri_body(i, _1):
            for b in range(3):
                slot(3 * i + b, b, False, True)
            return 0

        lax.fori_loop(1, (_SCCH - 1) // 3, tri_body, 0)
        slot(_SCCH - 1, 0, False, False)
        for b in range(3):
            wait_scatter(bufs[b][0], bufs[b][2])
        return 0

    lax.fori_loop(0, _NSC, sc_body, 0)
    plsc.subcore_barrier()

    pltpu.sync_copy(acc.at[pl.ds(s * _RPT, _RPT)],
                    part.at[c, pl.ds(s * _RPT, _RPT)])


def _combine_body(p_ref, t2_ref, th_ref, h_ref, h2_ref):
    h = 2.0 * (p_ref[0] + p_ref[1]) - t2_ref[...]
    h_ref[...] = h
    h2_ref[...] = th_ref[0, 0] * h


_combine = pl.pallas_call(
    _combine_body,
    grid=(_NRB,),
    in_specs=[
        pl.BlockSpec((_NC, _RB, _D), lambda i: (0, i, 0)),
        pl.BlockSpec((_RB, _D), lambda i: (i, 0)),
        pl.BlockSpec((8, _D), lambda i: (0, 0)),
    ],
    out_specs=[
        pl.BlockSpec((_RB, _D), lambda i: (i, 0)),
        pl.BlockSpec((_RB, _D), lambda i: (i, 0)),
    ],
    out_shape=(
        jax.ShapeDtypeStruct((_N, _D), jnp.float32),
        jax.ShapeDtypeStruct((_N, _D), jnp.float32),
    ),
)


def kernel(T_n_1, T_n_2, edge_index, edge_vals, theta):
    col = edge_index[1]
    row3 = jnp.reshape(edge_index[0], (_NW * _NSC, _SCCH, _CH))
    thb = jnp.broadcast_to(theta.astype(jnp.float32).reshape(1, 1), (8, _D))

    part = _spmm(T_n_1, col, row3, edge_vals)
    H_l, out2 = _combine(part, T_n_2, thb)
    return (H_l, out2)


# speedup vs baseline: 11.3683x; 1.0240x over previous
import functools

import jax
import jax.numpy as jnp
from jax import lax
from jax.experimental import pallas as pl
from jax.experimental.pallas import tpu as pltpu
from jax.experimental.pallas import tpu_sc as plsc

_N = 10000
_E = 320000
_D = 128
_NC = 2
_NS = 16
_NW = _NC * _NS
_L = 16
_EPT = _E // _NW
_CH = 80
_SCCH = 25
_ESC = _SCCH * _CH
_NSC = _EPT // _ESC
_NP = 10240
_RPT = _NP // _NS
_ZCH = 64
_NZCH = _RPT // _ZCH
_RB = 1000
_NRB = _N // _RB

_mesh = plsc.VectorSubcoreMesh(core_axis_name="c", subcore_axis_name="s")


@functools.partial(
    pl.kernel,
    mesh=_mesh,
    out_type=jax.ShapeDtypeStruct((_NC, _NP, _D), jnp.float32),
    scratch_types=[
        pltpu.VMEM((_ESC,), jnp.int32),
        pltpu.VMEM((_SCCH, _CH), jnp.int32),
        pltpu.VMEM((_ESC,), jnp.float32),
        pltpu.VMEM((_CH, _D), jnp.float32),
        pltpu.VMEM((_CH, _D), jnp.float32),
        pltpu.VMEM((_CH, _D), jnp.float32),
        pltpu.VMEM((_CH, _D), jnp.float32),
        pltpu.VMEM_SHARED((_NP, _D), jnp.float32),
        pltpu.SemaphoreType.DMA,
        pltpu.SemaphoreType.DMA,
        pltpu.SemaphoreType.DMA,
        pltpu.SemaphoreType.DMA,
        pltpu.SemaphoreType.DMA,
        pltpu.SemaphoreType.DMA,
        pltpu.SemaphoreType.DMA,
        pltpu.SemaphoreType.DMA,
    ],
)
def _spmm(t1, cols, row3, vals, part,
          colv, rowv, valv, g0, g1, g2, g3, acc,
          gs0, gs1, gs2, gs3, ss0, ss1, ss2, ss3):
    c = lax.axis_index("c")
    s = lax.axis_index("s")
    w = c * _NS + s

    def z_body(i, _):
        for k in range(_D // _L):
            g0[i, pl.ds(k * _L, _L)] = jnp.zeros((_L,), jnp.float32)
        return 0

    lax.fori_loop(0, _CH, z_body, 0)
    for rc in range(_RPT // _CH):
        pltpu.sync_copy(g0, acc.at[pl.ds(s * _RPT + rc * _CH, _CH)])
    plsc.subcore_barrier()

    bufs = ((g0, gs0, ss0), (g1, gs1, ss1), (g2, gs2, ss2), (g3, gs3, ss3))

    def start_gather(j, gb, sem):
        pltpu.async_copy(t1.at[colv.at[pl.ds(j * _CH, _CH)]], gb, sem)

    def wait_gather(gb, sem):
        pltpu.make_async_copy(t1.at[pl.ds(0, _CH)], gb, sem).wait()

    def start_scatter(gb, j, sem):
        pltpu.async_copy(gb, acc.at[rowv.at[j]], sem, add=True)

    def wait_scatter(gb, sem):
        pltpu.make_async_copy(gb, acc.at[pl.ds(0, _CH)], sem).wait()

    def scale(gb, j):
        def q_body(q, _2):
            g = q // 4
            vals16 = valv[pl.ds(j * _CH + g * _L, _L)]
            for d in range(4):
                lane = (q % 4) * 4 + d
                bc = lax.gather(
                    vals16,
                    jnp.full((_L, 1), lane, jnp.int32),
                    dimension_numbers=lax.GatherDimensionNumbers(
                        offset_dims=(), collapsed_slice_dims=(0,),
                        start_index_map=(0,)),
                    slice_sizes=(1,),
                    mode=lax.GatherScatterMode.PROMISE_IN_BOUNDS)
                r = q * 4 + d
                for k in range(_D // _L):
                    sl = pl.ds(k * _L, _L)
                    gb[r, sl] = gb[r, sl] * bc
            return 0

        lax.fori_loop(0, _CH // 4, q_body, 0)

    def slot(j, b, first, prefetch):
        buf, gs, ss = bufs[b]
        pbuf, pgs, pss = bufs[(b + 3) % 4]
        wait_gather(buf, gs)
        scale(buf, j)
        start_scatter(buf, j, ss)
        if prefetch:
            if first:
                start_gather(j + 3, pbuf, pgs)
            else:
                @pl.when(j + 3 < _SCCH)
                def _():
                    wait_scatter(pbuf, pss)
                    start_gather(j + 3, pbuf, pgs)

    def sc_body(t, _0):
        e0 = w * _EPT + t * _ESC
        pltpu.sync_copy(cols.at[pl.ds(e0, _ESC)], colv)
        pltpu.sync_copy(row3.at[w * _NSC + t], rowv)
        pltpu.sync_copy(vals.at[pl.ds(e0, _ESC)], valv)

        start_gather(0, g0, gs0)
        start_gather(1, g1, gs1)
        start_gather(2, g2, gs2)
        slot(0, 0, True, True)
        slot(1, 1, False, True)
        slot(2, 2, False, True)
        slot(3, 3, False, True)

        def quad_body(i, _1):
            for b in range(4):
                slot(4 * i + b, b, False, True)
            return 0

        lax.fori_loop(1, _SCCH // 4, quad_body, 0)
        slot(_SCCH - 1, 0, False, False)
        for b in range(4):
            wait_scatter(bufs[b][0], bufs[b][2])
        return 0

    lax.fori_loop(0, _NSC, sc_body, 0)
    plsc.subcore_barrier()

    pltpu.sync_copy(acc.at[pl.ds(s * _RPT, _RPT)],
                    part.at[c, pl.ds(s * _RPT, _RPT)])


def _combine_body(p_ref, t2_ref, th_ref, h_ref, h2_ref):
    h = 2.0 * (p_ref[0] + p_ref[1]) - t2_ref[...]
    h_ref[...] = h
    h2_ref[...] = th_ref[0, 0] * h


_combine = pl.pallas_call(
    _combine_body,
    grid=(_NRB,),
    in_specs=[
        pl.BlockSpec((_NC, _RB, _D), lambda i: (0, i, 0)),
        pl.BlockSpec((_RB, _D), lambda i: (i, 0)),
        pl.BlockSpec((8, _D), lambda i: (0, 0)),
    ],
    out_specs=[
        pl.BlockSpec((_RB, _D), lambda i: (i, 0)),
        pl.BlockSpec((_RB, _D), lambda i: (i, 0)),
    ],
    out_shape=(
        jax.ShapeDtypeStruct((_N, _D), jnp.float32),
        jax.ShapeDtypeStruct((_N, _D), jnp.float32),
    ),
)


def kernel(T_n_1, T_n_2, edge_index, edge_vals, theta):
    col = edge_index[1]
    row3 = jnp.reshape(edge_index[0], (_NW * _NSC, _SCCH, _CH))
    thb = jnp.broadcast_to(theta.astype(jnp.float32).reshape(1, 1), (8, _D))

    part = _spmm(T_n_1, col, row3, edge_vals)
    H_l, out2 = _combine(part, T_n_2, thb)
    return (H_l, out2)


# speedup vs baseline: 11.4462x; 1.0069x over previous
import functools

import jax
import jax.numpy as jnp
from jax import lax
from jax.experimental import pallas as pl
from jax.experimental.pallas import tpu as pltpu
from jax.experimental.pallas import tpu_sc as plsc

_N = 10000
_E = 320000
_D = 128
_NC = 2
_NS = 16
_NW = _NC * _NS
_L = 16
_EPT = _E // _NW
_CH = 80
_SCCH = 25
_ESC = _SCCH * _CH
_NSC = _EPT // _ESC
_NP = 10240
_RPT = _NP // _NS
_ZCH = 64
_NZCH = _RPT // _ZCH
_RB = 2000
_NRB = _N // _RB

_mesh = plsc.VectorSubcoreMesh(core_axis_name="c", subcore_axis_name="s")


@functools.partial(
    pl.kernel,
    mesh=_mesh,
    out_type=jax.ShapeDtypeStruct((_NC, _NP, _D), jnp.float32),
    scratch_types=[
        pltpu.VMEM((_ESC,), jnp.int32),
        pltpu.VMEM((_SCCH, _CH), jnp.int32),
        pltpu.VMEM((_ESC,), jnp.float32),
        pltpu.VMEM((_ESC,), jnp.int32),
        pltpu.VMEM((_SCCH, _CH), jnp.int32),
        pltpu.VMEM((_ESC,), jnp.float32),
        pltpu.VMEM((_CH, _D), jnp.float32),
        pltpu.VMEM((_CH, _D), jnp.float32),
        pltpu.VMEM((_CH, _D), jnp.float32),
        pltpu.VMEM_SHARED((_NP, _D), jnp.float32),
        pltpu.SemaphoreType.DMA,
        pltpu.SemaphoreType.DMA,
        pltpu.SemaphoreType.DMA,
        pltpu.SemaphoreType.DMA,
        pltpu.SemaphoreType.DMA,
        pltpu.SemaphoreType.DMA,
        pltpu.SemaphoreType.DMA,
        pltpu.SemaphoreType.DMA,
    ],
)
def _spmm(t1, cols, row3, vals, part,
          colv, rowv, valv, colv2, rowv2, valv2, g0, g1, g2, acc,
          gs0, gs1, gs2, ss0, ss1, ss2, sta, stb):
    c = lax.axis_index("c")
    s = lax.axis_index("s")
    w = c * _NS + s

    def z_body(i, _):
        for k in range(_D // _L):
            g0[i, pl.ds(k * _L, _L)] = jnp.zeros((_L,), jnp.float32)
        return 0

    lax.fori_loop(0, _CH, z_body, 0)
    for rc in range(_RPT // _CH):
        pltpu.sync_copy(g0, acc.at[pl.ds(s * _RPT + rc * _CH, _CH)])
    plsc.subcore_barrier()

    bufs = ((g0, gs0, ss0), (g1, gs1, ss1), (g2, gs2, ss2))

    def start_gather(cv, j, gb, sem):
        pltpu.async_copy(t1.at[cv.at[pl.ds(j * _CH, _CH)]], gb, sem)

    def wait_gather(gb, sem):
        pltpu.make_async_copy(t1.at[pl.ds(0, _CH)], gb, sem).wait()

    def start_scatter(rv, gb, j, sem):
        pltpu.async_copy(gb, acc.at[rv.at[j]], sem, add=True)

    def wait_scatter(gb, sem):
        pltpu.make_async_copy(gb, acc.at[pl.ds(0, _CH)], sem).wait()

    def scale(vv, gb, j):
        def q_body(q, _2):
            g = q // 4
            vals16 = vv[pl.ds(j * _CH + g * _L, _L)]
            for d in range(4):
                lane = (q % 4) * 4 + d
                bc = lax.gather(
                    vals16,
                    jnp.full((_L, 1), lane, jnp.int32),
                    dimension_numbers=lax.GatherDimensionNumbers(
                        offset_dims=(), collapsed_slice_dims=(0,),
                        start_index_map=(0,)),
                    slice_sizes=(1,),
                    mode=lax.GatherScatterMode.PROMISE_IN_BOUNDS)
                r = q * 4 + d
                for k in range(_D // _L):
                    sl = pl.ds(k * _L, _L)
                    gb[r, sl] = gb[r, sl] * bc
            return 0

        lax.fori_loop(0, _CH // 4, q_body, 0)

    def slot(cv, rv, vv, j, b, first, prefetch):
        buf, gs, ss = bufs[b]
        pbuf, pgs, pss = bufs[(b + 2) % 3]
        wait_gather(buf, gs)
        scale(vv, buf, j)
        start_scatter(rv, buf, j, ss)
        if prefetch:
            if first:
                start_gather(cv, j + 2, pbuf, pgs)
            else:
                @pl.when(j + 2 < _SCCH)
                def _():
                    wait_scatter(pbuf, pss)
                    start_gather(cv, j + 2, pbuf, pgs)

    def stage_sync(t, cv, rv, vv):
        e0 = w * _EPT + t * _ESC
        pltpu.sync_copy(cols.at[pl.ds(e0, _ESC)], cv)
        pltpu.sync_copy(row3.at[w * _NSC + t], rv)
        pltpu.sync_copy(vals.at[pl.ds(e0, _ESC)], vv)

    def stage_async(t, cv, rv, vv, sem):
        e0 = w * _EPT + t * _ESC
        pltpu.async_copy(cols.at[pl.ds(e0, _ESC)], cv, sem)
        pltpu.async_copy(row3.at[w * _NSC + t], rv, sem)
        pltpu.async_copy(vals.at[pl.ds(e0, _ESC)], vv, sem)

    def stage_wait(cv, rv, vv, sem):
        pltpu.make_async_copy(cols.at[pl.ds(0, _ESC)], cv, sem).wait()
        pltpu.make_async_copy(row3.at[0], rv, sem).wait()
        pltpu.make_async_copy(vals.at[pl.ds(0, _ESC)], vv, sem).wait()

    def process_sc(t, cv, rv, vv):
        start_gather(cv, 0, g0, gs0)
        start_gather(cv, 1, g1, gs1)
        slot(cv, rv, vv, 0, 0, True, True)
        slot(cv, rv, vv, 1, 1, False, True)
        slot(cv, rv, vv, 2, 2, False, True)

        def tri_body(i, _1):
            for b in range(3):
                slot(cv, rv, vv, 3 * i + b, b, False, True)
            return 0

        lax.fori_loop(1, (_SCCH - 1) // 3, tri_body, 0)
        slot(cv, rv, vv, _SCCH - 1, 0, False, False)
        for b in range(3):
            wait_scatter(bufs[b][0], bufs[b][2])

    stage_sync(0, colv, rowv, valv)
    stage_async(1, colv2, rowv2, valv2, stb)

    def tp_body(tp, _0):
        @pl.when(tp > 0)
        def _():
            stage_wait(colv, rowv, valv, sta)

        process_sc(2 * tp, colv, rowv, valv)

        @pl.when(2 * tp + 2 < _NSC)
        def _():
            stage_async(2 * tp + 2, colv, rowv, valv, sta)

        stage_wait(colv2, rowv2, valv2, stb)
        process_sc(2 * tp + 1, colv2, rowv2, valv2)

        @pl.when(2 * tp + 3 < _NSC)
        def _():
            stage_async(2 * tp + 3, colv2, rowv2, valv2, stb)
        return 0

    lax.fori_loop(0, (_NSC - 1) // 2, tp_body, 0)
    stage_wait(colv, rowv, valv, sta)
    process_sc(_NSC - 1, colv, rowv, valv)
    plsc.subcore_barrier()

    pltpu.sync_copy(acc.at[pl.ds(s * _RPT, _RPT)],
                    part.at[c, pl.ds(s * _RPT, _RPT)])


def _combine_body(p_ref, t2_ref, th_ref, h_ref, h2_ref):
    h = 2.0 * (p_ref[0] + p_ref[1]) - t2_ref[...]
    h_ref[...] = h
    h2_ref[...] = th_ref[0, 0] * h


_combine = pl.pallas_call(
    _combine_body,
    grid=(_NRB,),
    in_specs=[
        pl.BlockSpec((_NC, _RB, _D), lambda i: (0, i, 0)),
        pl.BlockSpec((_RB, _D), lambda i: (i, 0)),
        pl.BlockSpec((8, _D), lambda i: (0, 0)),
    ],
    out_specs=[
        pl.BlockSpec((_RB, _D), lambda i: (i, 0)),
        pl.BlockSpec((_RB, _D), lambda i: (i, 0)),
    ],
    out_shape=(
        jax.ShapeDtypeStruct((_N, _D), jnp.float32),
        jax.ShapeDtypeStruct((_N, _D), jnp.float32),
    ),
)


def kernel(T_n_1, T_n_2, edge_index, edge_vals, theta):
    col = edge_index[1]
    row3 = jnp.reshape(edge_index[0], (_NW * _NSC, _SCCH, _CH))
    thb = jnp.broadcast_to(theta.astype(jnp.float32).reshape(1, 1), (8, _D))

    part = _spmm(T_n_1, col, row3, edge_vals)
    H_l, out2 = _combine(part, T_n_2, thb)
    return (H_l, out2)


# speedup vs baseline: 11.4601x; 1.0012x over previous
import functools

import jax
import jax.numpy as jnp
from jax import lax
from jax.experimental import pallas as pl
from jax.experimental.pallas import tpu as pltpu
from jax.experimental.pallas import tpu_sc as plsc

_N = 10000
_E = 320000
_D = 128
_NC = 2
_NS = 16
_NW = _NC * _NS
_L = 16
_EPT = _E // _NW
_CH = 80
_SCCH = 25
_ESC = _SCCH * _CH
_NSC = _EPT // _ESC
_NP = 10240
_RPT = _NP // _NS
_ZCH = 64
_NZCH = _RPT // _ZCH
_RB = 2000
_NRB = _N // _RB

def _spmm_kernel():
    return pl.kernel(
        _spmm_body,
        mesh=plsc.VectorSubcoreMesh(core_axis_name="c", subcore_axis_name="s"),
        out_type=jax.ShapeDtypeStruct((_NC, _NP, _D), jnp.float32),
        scratch_types=[
            pltpu.VMEM((_ESC,), jnp.int32),
            pltpu.VMEM((_SCCH, _CH), jnp.int32),
            pltpu.VMEM((_ESC,), jnp.float32),
            pltpu.VMEM((_ESC,), jnp.int32),
            pltpu.VMEM((_SCCH, _CH), jnp.int32),
            pltpu.VMEM((_ESC,), jnp.float32),
            pltpu.VMEM((_CH, _D), jnp.float32),
            pltpu.VMEM((_CH, _D), jnp.float32),
            pltpu.VMEM((_CH, _D), jnp.float32),
            pltpu.VMEM_SHARED((_NP, _D), jnp.float32),
            pltpu.SemaphoreType.DMA,
            pltpu.SemaphoreType.DMA,
            pltpu.SemaphoreType.DMA,
            pltpu.SemaphoreType.DMA,
            pltpu.SemaphoreType.DMA,
            pltpu.SemaphoreType.DMA,
            pltpu.SemaphoreType.DMA,
            pltpu.SemaphoreType.DMA,
        ],
    )


def _spmm_body(t1, cols, row3, vals, part,
          colv, rowv, valv, colv2, rowv2, valv2, g0, g1, g2, acc,
          gs0, gs1, gs2, ss0, ss1, ss2, sta, stb):
    c = lax.axis_index("c")
    s = lax.axis_index("s")
    w = c * _NS + s

    def z_body(i, _):
        for k in range(_D // _L):
            g0[i, pl.ds(k * _L, _L)] = jnp.zeros((_L,), jnp.float32)
        return 0

    lax.fori_loop(0, _CH, z_body, 0)
    for rc in range(_RPT // _CH):
        pltpu.sync_copy(g0, acc.at[pl.ds(s * _RPT + rc * _CH, _CH)])
    plsc.subcore_barrier()

    bufs = ((g0, gs0, ss0), (g1, gs1, ss1), (g2, gs2, ss2))

    def start_gather(cv, j, gb, sem):
        pltpu.async_copy(t1.at[cv.at[pl.ds(j * _CH, _CH)]], gb, sem)

    def wait_gather(gb, sem):
        pltpu.make_async_copy(t1.at[pl.ds(0, _CH)], gb, sem).wait()

    def start_scatter(rv, gb, j, sem):
        pltpu.async_copy(gb, acc.at[rv.at[j]], sem, add=True)

    def wait_scatter(gb, sem):
        pltpu.make_async_copy(gb, acc.at[pl.ds(0, _CH)], sem).wait()

    def scale(vv, gb, j):
        def q_body(q, _2):
            g = q // 4
            vals16 = vv[pl.ds(j * _CH + g * _L, _L)]
            for d in range(4):
                lane = (q % 4) * 4 + d
                bc = lax.gather(
                    vals16,
                    jnp.full((_L, 1), lane, jnp.int32),
                    dimension_numbers=lax.GatherDimensionNumbers(
                        offset_dims=(), collapsed_slice_dims=(0,),
                        start_index_map=(0,)),
                    slice_sizes=(1,),
                    mode=lax.GatherScatterMode.PROMISE_IN_BOUNDS)
                r = q * 4 + d
                for k in range(_D // _L):
                    sl = pl.ds(k * _L, _L)
                    gb[r, sl] = gb[r, sl] * bc
            return 0

        lax.fori_loop(0, _CH // 4, q_body, 0)

    def slot(cv, rv, vv, j, b, first, prefetch):
        buf, gs, ss = bufs[b]
        pbuf, pgs, pss = bufs[(b + 2) % 3]
        wait_gather(buf, gs)
        scale(vv, buf, j)
        start_scatter(rv, buf, j, ss)
        if prefetch:
            if first:
                start_gather(cv, j + 2, pbuf, pgs)
            else:
                @pl.when(j + 2 < _SCCH)
                def _():
                    wait_scatter(pbuf, pss)
                    start_gather(cv, j + 2, pbuf, pgs)

    def stage_sync(t, cv, rv, vv):
        e0 = w * _EPT + t * _ESC
        pltpu.sync_copy(cols.at[pl.ds(e0, _ESC)], cv)
        pltpu.sync_copy(row3.at[w * _NSC + t], rv)
        pltpu.sync_copy(vals.at[pl.ds(e0, _ESC)], vv)

    def stage_async(t, cv, rv, vv, sem):
        e0 = w * _EPT + t * _ESC
        pltpu.async_copy(cols.at[pl.ds(e0, _ESC)], cv, sem)
        pltpu.async_copy(row3.at[w * _NSC + t], rv, sem)
        pltpu.async_copy(vals.at[pl.ds(e0, _ESC)], vv, sem)

    def stage_wait(cv, rv, vv, sem):
        pltpu.make_async_copy(cols.at[pl.ds(0, _ESC)], cv, sem).wait()
        pltpu.make_async_copy(row3.at[0], rv, sem).wait()
        pltpu.make_async_copy(vals.at[pl.ds(0, _ESC)], vv, sem).wait()

    def process_sc(t, cv, rv, vv):
        start_gather(cv, 0, g0, gs0)
        start_gather(cv, 1, g1, gs1)
        slot(cv, rv, vv, 0, 0, True, True)
        slot(cv, rv, vv, 1, 1, False, True)
        slot(cv, rv, vv, 2, 2, False, True)

        def tri_body(i, _1):
            for b in range(3):
                slot(cv, rv, vv, 3 * i + b, b, False, True)
            return 0

        lax.fori_loop(1, (_SCCH - 1) // 3, tri_body, 0)
        slot(cv, rv, vv, _SCCH - 1, 0, False, False)
        for b in range(3):
            wait_scatter(bufs[b][0], bufs[b][2])

    stage_sync(0, colv, rowv, valv)
    stage_async(1, colv2, rowv2, valv2, stb)

    def tp_body(tp, _0):
        @pl.when(tp > 0)
        def _():
            stage_wait(colv, rowv, valv, sta)

        process_sc(2 * tp, colv, rowv, valv)

        @pl.when(2 * tp + 2 < _NSC)
        def _():
            stage_async(2 * tp + 2, colv, rowv, valv, sta)

        stage_wait(colv2, rowv2, valv2, stb)
        process_sc(2 * tp + 1, colv2, rowv2, valv2)

        @pl.when(2 * tp + 3 < _NSC)
        def _():
            stage_async(2 * tp + 3, colv2, rowv2, valv2, stb)
        return 0

    lax.fori_loop(0, (_NSC - 1) // 2, tp_body, 0)
    stage_wait(colv, rowv, valv, sta)
    process_sc(_NSC - 1, colv, rowv, valv)
    plsc.subcore_barrier()

    pltpu.sync_copy(acc.at[pl.ds(s * _RPT, _RPT)],
                    part.at[c, pl.ds(s * _RPT, _RPT)])


def _combine_body(p_ref, t2_ref, th_ref, h_ref, h2_ref):
    h = 2.0 * (p_ref[0] + p_ref[1]) - t2_ref[...]
    h_ref[...] = h
    h2_ref[...] = th_ref[0, 0] * h


_combine = pl.pallas_call(
    _combine_body,
    grid=(_NRB,),
    in_specs=[
        pl.BlockSpec((_NC, _RB, _D), lambda i: (0, i, 0)),
        pl.BlockSpec((_RB, _D), lambda i: (i, 0)),
        pl.BlockSpec((8, _D), lambda i: (0, 0)),
    ],
    out_specs=[
        pl.BlockSpec((_RB, _D), lambda i: (i, 0)),
        pl.BlockSpec((_RB, _D), lambda i: (i, 0)),
    ],
    out_shape=(
        jax.ShapeDtypeStruct((_N, _D), jnp.float32),
        jax.ShapeDtypeStruct((_N, _D), jnp.float32),
    ),
)


def kernel(T_n_1, T_n_2, edge_index, edge_vals, theta):
    col = edge_index[1]
    row3 = jnp.reshape(edge_index[0], (_NW * _NSC, _SCCH, _CH))
    thb = jnp.broadcast_to(theta.astype(jnp.float32).reshape(1, 1), (8, _D))

    part = _spmm_kernel()(T_n_1, col, row3, edge_vals)
    H_l, out2 = _combine(part, T_n_2, thb)
    return (H_l, out2)
